# Initial kernel scaffold; baseline (speedup 1.0000x reference)
import jax
import jax.numpy as jnp
from jax.experimental import pallas as pl


def kernel(x, edge_index, W_in, b_in, conv_W, conv_b, conv_gamma, conv_beta, att1_Wq, att1_bq, att1_Wk, att1_bk, att1_Wv, att1_bv, att1_Wo, att1_bo, att1_g, att1_be, att2_Wq, att2_bq, att2_Wk, att2_bk, att2_Wv, att2_bv, att2_Wo, att2_bo, att2_g, att2_be, se_W1, se_b1, se_W2, se_b2):
    raise NotImplementedError("write your pallas kernel here")


# speedup vs baseline: 10.2026x; 10.2026x over previous
import functools

import jax
import jax.numpy as jnp
import numpy as np
from jax import lax
from jax.experimental import pallas as pl
from jax.experimental.pallas import tpu as pltpu
from jax.experimental.pallas import tpu_sc as plsc

_B, _N, _D, _E, _H = 16, 512, 256, 8192, 4
_DH = _D // _H
_NC, _NS = 2, 16
_NW = _NC * _NS
_EPW = _E // _NW
_ACC = 2 * _N * _N
_ACC_PER_SUB = _ACC // _NS


def _adj_counts_sc(edge_index, zeros_acc):
    mesh = plsc.VectorSubcoreMesh(core_axis_name="c", subcore_axis_name="s")

    @functools.partial(
        pl.kernel,
        mesh=mesh,
        out_type=jax.ShapeDtypeStruct((_NC, _ACC), jnp.float32),
        scratch_types=[
            pltpu.VMEM((_EPW,), jnp.int32),
            pltpu.VMEM((_EPW,), jnp.int32),
            pltpu.VMEM((4, 128), jnp.int32),
            pltpu.VMEM((128,), jnp.float32),
            pltpu.VMEM_SHARED((_ACC,), jnp.float32),
        ],
    )
    def _k(edge_hbm, zeros_hbm, out_hbm, row_v, col_v, idx_v, ones_v, acc_sh):
        c = lax.axis_index("c")
        s = lax.axis_index("s")
        wid = c * _NS + s
        zbase = s * _ACC_PER_SUB
        pltpu.sync_copy(zeros_hbm.at[pl.ds(zbase, _ACC_PER_SUB)],
                        acc_sh.at[pl.ds(zbase, _ACC_PER_SUB)])
        ebase = wid * _EPW
        pltpu.sync_copy(edge_hbm.at[0, pl.ds(ebase, _EPW)], row_v)
        pltpu.sync_copy(edge_hbm.at[1, pl.ds(ebase, _EPW)], col_v)
        for i in range(128 // 16):
            ones_v[pl.ds(i * 16, 16)] = jnp.ones((16,), jnp.float32)
        for j in range(2):
            for i in range(128 // 16):
                sl = pl.ds(j * 128 + i * 16, 16)
                r = row_v[sl]
                cc = col_v[sl]
                dst = pl.ds(i * 16, 16)
                idx_v[j, dst] = cc * _N + r
                idx_v[j + 2, dst] = (_N * _N) + r * _N + cc
        plsc.subcore_barrier()
        for j in range(4):
            pltpu.sync_copy(ones_v, acc_sh.at[idx_v.at[j]], add=True)
        plsc.subcore_barrier()
        pltpu.sync_copy(acc_sh.at[pl.ds(zbase, _ACC_PER_SUB)],
                        out_hbm.at[c, pl.ds(zbase, _ACC_PER_SUB)])

    return _k(edge_index, zeros_acc)


def _attention(h, bias, Wq, bq, Wk, bk, Wv, bv, Wo, bo, g, be):
    f32 = jnp.float32
    Q = jnp.dot(h, Wq, preferred_element_type=f32) + bq
    K = jnp.dot(h, Wk, preferred_element_type=f32) + bk
    V = jnp.dot(h, Wv, preferred_element_type=f32) + bv
    scale = _DH ** -0.5
    outs = []
    for hd in range(_H):
        cs = slice(hd * _DH, (hd + 1) * _DH)
        S = lax.dot_general(Q[:, cs], K[:, cs], (((1,), (1,)), ((), ())),
                            preferred_element_type=f32) * scale + bias
        m = jnp.max(S, axis=1, keepdims=True)
        e = jnp.exp(S - m)
        w = e / jnp.sum(e, axis=1, keepdims=True)
        outs.append(jnp.dot(w, V[:, cs], preferred_element_type=f32))
    att = jnp.concatenate(outs, axis=1)
    o = jnp.dot(att, Wo, preferred_element_type=f32) + bo
    x = o + h
    mu = jnp.mean(x, axis=1, keepdims=True)
    var = jnp.mean((x - mu) ** 2, axis=1, keepdims=True)
    return (x - mu) * lax.rsqrt(var + 1e-5) * g + be


def _tc_body(x_ref, adj_ref, W_in_ref, b_in_ref, conv_W_ref, conv_b_ref,
             cg_ref, cb_ref,
             q1_ref, bq1_ref, k1_ref, bk1_ref, v1_ref, bv1_ref, o1_ref,
             bo1_ref, g1_ref, be1_ref,
             q2_ref, bq2_ref, k2_ref, bk2_ref, v2_ref, bv2_ref, o2_ref,
             bo2_ref, g2_ref, be2_ref,
             sw1_ref, sb1_ref, sw2_ref, sb2_ref, out_ref):
    f32 = jnp.float32
    xb = x_ref[0]
    A = adj_ref[0, 0] + adj_ref[1, 0]
    Mc = adj_ref[0, 1] + adj_ref[1, 1]
    h = jnp.dot(xb, W_in_ref[...], preferred_element_type=f32) + b_in_ref[...]
    t = jnp.dot(h, conv_W_ref[...], preferred_element_type=f32) + conv_b_ref[...]
    agg = jnp.dot(A, t, preferred_element_type=f32)
    deg = jnp.maximum(jnp.sum(Mc, axis=1, keepdims=True), 1.0)
    out = agg / deg + t
    out = out * (cg_ref[...] * np.float32(1.0 / np.sqrt(1.0 + 1e-5))) + cb_ref[...]
    h = jnp.maximum(out, 0.0)
    ri = lax.broadcasted_iota(jnp.int32, (_N, _N), 0)
    ci = lax.broadcasted_iota(jnp.int32, (_N, _N), 1)
    allowed = (Mc > 0.0) | (ri == ci)
    bias = jnp.where(allowed, 0.0, -1e30)
    h = _attention(h, bias, q1_ref[...], bq1_ref[...], k1_ref[...],
                   bk1_ref[...], v1_ref[...], bv1_ref[...], o1_ref[...],
                   bo1_ref[...], g1_ref[...], be1_ref[...])
    h = _attention(h, bias, q2_ref[...], bq2_ref[...], k2_ref[...],
                   bk2_ref[...], v2_ref[...], bv2_ref[...], o2_ref[...],
                   bo2_ref[...], g2_ref[...], be2_ref[...])
    mu = jnp.mean(h, axis=1, keepdims=True)
    sq = jnp.sum((h - mu) ** 2, axis=1, keepdims=True) / np.float32(_D - 1)
    e1 = lax.dot_general(sw1_ref[...], sq, (((0,), (0,)), ((), ())),
                         preferred_element_type=f32) + sb1_ref[...]
    e1 = jnp.maximum(e1, 0.0)
    e2 = lax.dot_general(sw2_ref[...], e1, (((0,), (0,)), ((), ())),
                         preferred_element_type=f32) + sb2_ref[...]
    ex = 1.0 / (1.0 + jnp.exp(-e2))
    out_ref[0] = h * ex


def _full(shape):
    return pl.BlockSpec(shape, lambda b: (0,) * len(shape))


def kernel(x, edge_index, W_in, b_in, conv_W, conv_b, conv_gamma, conv_beta,
           att1_Wq, att1_bq, att1_Wk, att1_bk, att1_Wv, att1_bv, att1_Wo,
           att1_bo, att1_g, att1_be, att2_Wq, att2_bq, att2_Wk, att2_bk,
           att2_Wv, att2_bv, att2_Wo, att2_bo, att2_g, att2_be,
           se_W1, se_b1, se_W2, se_b2):
    zeros_acc = jnp.zeros((_ACC,), jnp.float32)
    adj = _adj_counts_sc(edge_index, zeros_acc)
    adj = adj.reshape(_NC, 2, _N, _N)

    row = lambda v: v.reshape(1, -1)
    col = lambda v: v.reshape(-1, 1)
    dense = pl.pallas_call(
        _tc_body,
        grid=(_B,),
        in_specs=[
            pl.BlockSpec((1, _N, _D), lambda b: (b, 0, 0)),
            _full((_NC, 2, _N, _N)),
            _full((_D, _D)), _full((1, _D)),
            _full((_D, _D)), _full((1, _D)),
            _full((1, _D)), _full((1, _D)),
            _full((_D, _D)), _full((1, _D)),
            _full((_D, _D)), _full((1, _D)),
            _full((_D, _D)), _full((1, _D)),
            _full((_D, _D)), _full((1, _D)),
            _full((1, _D)), _full((1, _D)),
            _full((_D, _D)), _full((1, _D)),
            _full((_D, _D)), _full((1, _D)),
            _full((_D, _D)), _full((1, _D)),
            _full((_D, _D)), _full((1, _D)),
            _full((1, _D)), _full((1, _D)),
            _full((_N, _N // 4)), _full((_N // 4, 1)),
            _full((_N // 4, _N)), _full((_N, 1)),
        ],
        out_specs=pl.BlockSpec((1, _N, _D), lambda b: (b, 0, 0)),
        out_shape=jax.ShapeDtypeStruct((_B, _N, _D), jnp.float32),
    )
    return dense(
        x, adj,
        W_in, row(b_in), conv_W, row(conv_b), row(conv_gamma), row(conv_beta),
        att1_Wq, row(att1_bq), att1_Wk, row(att1_bk), att1_Wv, row(att1_bv),
        att1_Wo, row(att1_bo), row(att1_g), row(att1_be),
        att2_Wq, row(att2_bq), att2_Wk, row(att2_bk), att2_Wv, row(att2_bv),
        att2_Wo, row(att2_bo), row(att2_g), row(att2_be),
        se_W1, col(se_b1), se_W2, col(se_b2),
    )
